# baseline (speedup 1.0000x reference)
import functools

import jax
import jax.numpy as jnp
from jax import lax
from jax.experimental import pallas as pl
from jax.experimental.pallas import tpu as pltpu
from jax.experimental.pallas import tpu_sc as plsc

VOCAB = 1000000
EMBED_DIM = 32
CTX = 200

NUM_WORKERS = 32
ROWS_PER_WORKER = 8
VALID_WORKERS = CTX // ROWS_PER_WORKER

VB = 32768
NBLK = (VOCAB + VB - 1) // VB
VPAD = NBLK * VB
ROWS2D = VPAD // 128

VB3 = 131072
NBLK3 = (VOCAB + VB3 - 1) // VB3

NEG_BIG = -3.4e38


def _sc_gather_body(idx_hbm, table_hbm, out_hbm, idx_v, row_v, acc_v, sem):
    wid = lax.axis_index("s") * 2 + lax.axis_index("c")
    base = wid * ROWS_PER_WORKER
    pltpu.sync_copy(idx_hbm.at[pl.ds(base, ROWS_PER_WORKER)],
                    idx_v.at[pl.ds(0, ROWS_PER_WORKER)])
    idx_vec = idx_v[...]
    acc0 = jnp.zeros((16,), jnp.float32)
    acc1 = jnp.zeros((16,), jnp.float32)
    for j in range(ROWS_PER_WORKER):
        r = idx_vec[j]
        pltpu.sync_copy(table_hbm.at[pl.ds(r, 1), :], row_v)
        acc0 = acc0 + row_v[0, pl.ds(0, 16)]
        acc1 = acc1 + row_v[0, pl.ds(16, 16)]
    valid = wid < VALID_WORKERS
    acc0 = jnp.where(valid, acc0, jnp.zeros((16,), jnp.float32))
    acc1 = jnp.where(valid, acc1, jnp.zeros((16,), jnp.float32))
    acc_v[pl.ds(0, 16)] = acc0
    acc_v[pl.ds(16, 16)] = acc1
    pltpu.sync_copy(acc_v, out_hbm.at[wid])


_SC_GATHER_CACHE = []


def _sc_gather(idx, table):
    if not _SC_GATHER_CACHE:
        _SC_GATHER_CACHE.append(functools.partial(
            pl.kernel,
            mesh=plsc.VectorSubcoreMesh(core_axis_name="c", subcore_axis_name="s"),
            out_type=jax.ShapeDtypeStruct((NUM_WORKERS, EMBED_DIM), jnp.float32),
            scratch_types=[
                pltpu.VMEM((16,), jnp.int32),
                pltpu.VMEM((1, EMBED_DIM), jnp.float32),
                pltpu.VMEM((EMBED_DIM,), jnp.float32),
                pltpu.SemaphoreType.DMA,
            ],
        )(_sc_gather_body))
    return _SC_GATHER_CACHE[0](idx, table)


def _gather_body(idx_ref, table_t_ref, out_ref, blocks_ref, sem):
    copies = []
    for j in range(CTX):
        base = (idx_ref[j] // 128) * 128
        c = pltpu.make_async_copy(
            table_t_ref.at[:, pl.ds(base, 128)],
            blocks_ref.at[j],
            sem,
        )
        c.start()
        copies.append(c)
    for c in copies:
        c.wait()
    lane = lax.broadcasted_iota(jnp.int32, (EMBED_DIM, 128), 1)
    acc = jnp.zeros((EMBED_DIM, 128), jnp.float32)
    for j in range(CTX):
        acc = acc + jnp.where(lane == idx_ref[j] % 128,
                              blocks_ref[j], 0.0)
    out_ref[...] = jnp.sum(acc, axis=1, keepdims=True)


def _tc_gather(idx, table_t):
    return pl.pallas_call(
        _gather_body,
        in_specs=[
            pl.BlockSpec(memory_space=pltpu.SMEM),
            pl.BlockSpec(memory_space=pl.ANY),
        ],
        out_specs=pl.BlockSpec((EMBED_DIM, 1), lambda: (0, 0)),
        out_shape=jax.ShapeDtypeStruct((EMBED_DIM, 1), jnp.float32),
        scratch_shapes=[
            pltpu.VMEM((CTX, EMBED_DIM, 128), jnp.float32),
            pltpu.SemaphoreType.DMA,
        ],
    )(idx, table_t)


def _logits_body(partials_ref, wt_ref, b_ref, out_ref):
    emb_col = partials_ref[...]
    prod = wt_ref[...] * emb_col
    out_ref[...] = jnp.sum(prod, axis=0) + b_ref[...]


def _logz_body(x_ref, logz_ref):
    x = x_ref[...]
    gidx = (lax.broadcasted_iota(jnp.int32, (ROWS2D, 128), 0) * 128
            + lax.broadcasted_iota(jnp.int32, (ROWS2D, 128), 1))
    x = jnp.where(gidx < VOCAB, x, NEG_BIG)
    m = jnp.max(x)
    s = jnp.sum(jnp.exp(x - m))
    logz_ref[0, 0] = m + jnp.log(s)


def _sub_body(logits_ref, logz_ref, out_ref):
    out_ref[...] = logits_ref[...] - logz_ref[0, 0]


def _tc_call(partials, wt, b):
    logits = pl.pallas_call(
        _logits_body,
        grid=(NBLK,),
        in_specs=[
            pl.BlockSpec((EMBED_DIM, 1), lambda i: (0, 0)),
            pl.BlockSpec((EMBED_DIM, VB), lambda i: (0, i)),
            pl.BlockSpec((VB,), lambda i: (i,)),
        ],
        out_specs=pl.BlockSpec((VB,), lambda i: (i,)),
        out_shape=jax.ShapeDtypeStruct((VPAD,), jnp.float32),
    )(partials, wt, b)
    logz = pl.pallas_call(
        _logz_body,
        in_specs=[pl.BlockSpec((ROWS2D, 128), lambda: (0, 0))],
        out_specs=pl.BlockSpec(memory_space=pltpu.SMEM),
        out_shape=jax.ShapeDtypeStruct((1, 1), jnp.float32),
    )(logits.reshape(ROWS2D, 128))
    return pl.pallas_call(
        _sub_body,
        grid=(NBLK3,),
        in_specs=[
            pl.BlockSpec((VB3,), lambda i: (i,)),
            pl.BlockSpec(memory_space=pltpu.SMEM),
        ],
        out_specs=pl.BlockSpec((VB3,), lambda i: (i,)),
        out_shape=jax.ShapeDtypeStruct((VOCAB,), jnp.float32),
    )(logits, logz)


def kernel(inputs, emb_table, W, b):
    partials = _tc_gather(inputs.astype(jnp.int32), emb_table.T)
    out = _tc_call(partials, W.T, b)
    return out.reshape(1, VOCAB)


# speedup vs baseline: 1.0767x; 1.0767x over previous
import functools

import jax
import jax.numpy as jnp
from jax import lax
from jax.experimental import pallas as pl
from jax.experimental.pallas import tpu as pltpu
from jax.experimental.pallas import tpu_sc as plsc

VOCAB = 1000000
EMBED_DIM = 32
CTX = 200

NUM_WORKERS = 32
ROWS_PER_WORKER = 8
VALID_WORKERS = CTX // ROWS_PER_WORKER

VB = 65536
NBLK = (VOCAB + VB - 1) // VB
VPAD = NBLK * VB
ROWS2D = VPAD // 128

VB3 = 131072
NBLK3 = (VOCAB + VB3 - 1) // VB3

NEG_BIG = -3.4e38


def _sc_gather_body(idx_hbm, table_hbm, out_hbm, idx_v, row_v, acc_v, sem):
    wid = lax.axis_index("s") * 2 + lax.axis_index("c")
    base = wid * ROWS_PER_WORKER
    pltpu.sync_copy(idx_hbm.at[pl.ds(base, ROWS_PER_WORKER)],
                    idx_v.at[pl.ds(0, ROWS_PER_WORKER)])
    idx_vec = idx_v[...]
    acc0 = jnp.zeros((16,), jnp.float32)
    acc1 = jnp.zeros((16,), jnp.float32)
    for j in range(ROWS_PER_WORKER):
        r = idx_vec[j]
        pltpu.sync_copy(table_hbm.at[pl.ds(r, 1), :], row_v)
        acc0 = acc0 + row_v[0, pl.ds(0, 16)]
        acc1 = acc1 + row_v[0, pl.ds(16, 16)]
    valid = wid < VALID_WORKERS
    acc0 = jnp.where(valid, acc0, jnp.zeros((16,), jnp.float32))
    acc1 = jnp.where(valid, acc1, jnp.zeros((16,), jnp.float32))
    acc_v[pl.ds(0, 16)] = acc0
    acc_v[pl.ds(16, 16)] = acc1
    pltpu.sync_copy(acc_v, out_hbm.at[wid])


_SC_GATHER_CACHE = []


def _sc_gather(idx, table):
    if not _SC_GATHER_CACHE:
        _SC_GATHER_CACHE.append(functools.partial(
            pl.kernel,
            mesh=plsc.VectorSubcoreMesh(core_axis_name="c", subcore_axis_name="s"),
            out_type=jax.ShapeDtypeStruct((NUM_WORKERS, EMBED_DIM), jnp.float32),
            scratch_types=[
                pltpu.VMEM((16,), jnp.int32),
                pltpu.VMEM((1, EMBED_DIM), jnp.float32),
                pltpu.VMEM((EMBED_DIM,), jnp.float32),
                pltpu.SemaphoreType.DMA,
            ],
        )(_sc_gather_body))
    return _SC_GATHER_CACHE[0](idx, table)


def _gather_body(idx_ref, table_t_ref, out_ref, blocks_ref, sem):
    copies = []
    for j in range(CTX):
        base = (idx_ref[j] // 128) * 128
        c = pltpu.make_async_copy(
            table_t_ref.at[:, pl.ds(base, 128)],
            blocks_ref.at[j],
            sem,
        )
        c.start()
        copies.append(c)
    for c in copies:
        c.wait()
    lane = lax.broadcasted_iota(jnp.int32, (EMBED_DIM, 128), 1)
    acc = jnp.zeros((EMBED_DIM, 128), jnp.float32)
    for j in range(CTX):
        acc = acc + jnp.where(lane == idx_ref[j] % 128,
                              blocks_ref[j], 0.0)
    out_ref[...] = jnp.sum(acc, axis=1, keepdims=True)


def _tc_gather(idx, table_t):
    return pl.pallas_call(
        _gather_body,
        in_specs=[
            pl.BlockSpec(memory_space=pltpu.SMEM),
            pl.BlockSpec(memory_space=pl.ANY),
        ],
        out_specs=pl.BlockSpec((EMBED_DIM, 1), lambda: (0, 0)),
        out_shape=jax.ShapeDtypeStruct((EMBED_DIM, 1), jnp.float32),
        scratch_shapes=[
            pltpu.VMEM((CTX, EMBED_DIM, 128), jnp.float32),
            pltpu.SemaphoreType.DMA,
        ],
    )(idx, table_t)


def _logits_body(partials_ref, wt_ref, b_ref, out_ref):
    emb_col = partials_ref[...]
    prod = wt_ref[...] * emb_col
    out_ref[...] = jnp.sum(prod, axis=0) + b_ref[...]


def _logz_body(x_ref, logz_ref):
    x = x_ref[...]
    gidx = (lax.broadcasted_iota(jnp.int32, (ROWS2D, 128), 0) * 128
            + lax.broadcasted_iota(jnp.int32, (ROWS2D, 128), 1))
    x = jnp.where(gidx < VOCAB, x, NEG_BIG)
    m = jnp.max(x)
    s = jnp.sum(jnp.exp(x - m))
    logz_ref[0, 0] = m + jnp.log(s)


def _sub_body(logits_ref, logz_ref, out_ref):
    out_ref[...] = logits_ref[...] - logz_ref[0, 0]


def _tc_call(partials, wt, b):
    logits = pl.pallas_call(
        _logits_body,
        grid=(NBLK,),
        in_specs=[
            pl.BlockSpec((EMBED_DIM, 1), lambda i: (0, 0)),
            pl.BlockSpec((EMBED_DIM, VB), lambda i: (0, i)),
            pl.BlockSpec((VB,), lambda i: (i,)),
        ],
        out_specs=pl.BlockSpec((VB,), lambda i: (i,)),
        out_shape=jax.ShapeDtypeStruct((VPAD,), jnp.float32),
    )(partials, wt, b)
    logz = pl.pallas_call(
        _logz_body,
        in_specs=[pl.BlockSpec((ROWS2D, 128), lambda: (0, 0))],
        out_specs=pl.BlockSpec(memory_space=pltpu.SMEM),
        out_shape=jax.ShapeDtypeStruct((1, 1), jnp.float32),
    )(logits.reshape(ROWS2D, 128))
    return pl.pallas_call(
        _sub_body,
        grid=(NBLK3,),
        in_specs=[
            pl.BlockSpec((VB3,), lambda i: (i,)),
            pl.BlockSpec(memory_space=pltpu.SMEM),
        ],
        out_specs=pl.BlockSpec((VB3,), lambda i: (i,)),
        out_shape=jax.ShapeDtypeStruct((VOCAB,), jnp.float32),
    )(logits, logz)


def kernel(inputs, emb_table, W, b):
    partials = _tc_gather(inputs.astype(jnp.int32), emb_table.T)
    out = _tc_call(partials, W.T, b)
    return out.reshape(1, VOCAB)


# speedup vs baseline: 1.0863x; 1.0089x over previous
import functools

import jax
import jax.numpy as jnp
from jax import lax
from jax.experimental import pallas as pl
from jax.experimental.pallas import tpu as pltpu
from jax.experimental.pallas import tpu_sc as plsc

VOCAB = 1000000
EMBED_DIM = 32
CTX = 200

NUM_WORKERS = 32
ROWS_PER_WORKER = 8
VALID_WORKERS = CTX // ROWS_PER_WORKER

VB = 131072
NBLK = (VOCAB + VB - 1) // VB
VPAD = NBLK * VB
ROWS2D = VPAD // 128

VB3 = 131072
NBLK3 = (VOCAB + VB3 - 1) // VB3

NEG_BIG = -3.4e38


def _sc_gather_body(idx_hbm, table_hbm, out_hbm, idx_v, row_v, acc_v, sem):
    wid = lax.axis_index("s") * 2 + lax.axis_index("c")
    base = wid * ROWS_PER_WORKER
    pltpu.sync_copy(idx_hbm.at[pl.ds(base, ROWS_PER_WORKER)],
                    idx_v.at[pl.ds(0, ROWS_PER_WORKER)])
    idx_vec = idx_v[...]
    acc0 = jnp.zeros((16,), jnp.float32)
    acc1 = jnp.zeros((16,), jnp.float32)
    for j in range(ROWS_PER_WORKER):
        r = idx_vec[j]
        pltpu.sync_copy(table_hbm.at[pl.ds(r, 1), :], row_v)
        acc0 = acc0 + row_v[0, pl.ds(0, 16)]
        acc1 = acc1 + row_v[0, pl.ds(16, 16)]
    valid = wid < VALID_WORKERS
    acc0 = jnp.where(valid, acc0, jnp.zeros((16,), jnp.float32))
    acc1 = jnp.where(valid, acc1, jnp.zeros((16,), jnp.float32))
    acc_v[pl.ds(0, 16)] = acc0
    acc_v[pl.ds(16, 16)] = acc1
    pltpu.sync_copy(acc_v, out_hbm.at[wid])


_SC_GATHER_CACHE = []


def _sc_gather(idx, table):
    if not _SC_GATHER_CACHE:
        _SC_GATHER_CACHE.append(functools.partial(
            pl.kernel,
            mesh=plsc.VectorSubcoreMesh(core_axis_name="c", subcore_axis_name="s"),
            out_type=jax.ShapeDtypeStruct((NUM_WORKERS, EMBED_DIM), jnp.float32),
            scratch_types=[
                pltpu.VMEM((16,), jnp.int32),
                pltpu.VMEM((1, EMBED_DIM), jnp.float32),
                pltpu.VMEM((EMBED_DIM,), jnp.float32),
                pltpu.SemaphoreType.DMA,
            ],
        )(_sc_gather_body))
    return _SC_GATHER_CACHE[0](idx, table)


def _gather_body(idx_ref, table_t_ref, out_ref, blocks_ref, sem):
    copies = []
    for j in range(CTX):
        base = (idx_ref[j] // 128) * 128
        c = pltpu.make_async_copy(
            table_t_ref.at[:, pl.ds(base, 128)],
            blocks_ref.at[j],
            sem,
        )
        c.start()
        copies.append(c)
    for c in copies:
        c.wait()
    lane = lax.broadcasted_iota(jnp.int32, (EMBED_DIM, 128), 1)
    acc = jnp.zeros((EMBED_DIM, 128), jnp.float32)
    for j in range(CTX):
        acc = acc + jnp.where(lane == idx_ref[j] % 128,
                              blocks_ref[j], 0.0)
    out_ref[...] = jnp.sum(acc, axis=1, keepdims=True)


def _tc_gather(idx, table_t):
    return pl.pallas_call(
        _gather_body,
        in_specs=[
            pl.BlockSpec(memory_space=pltpu.SMEM),
            pl.BlockSpec(memory_space=pl.ANY),
        ],
        out_specs=pl.BlockSpec((EMBED_DIM, 1), lambda: (0, 0)),
        out_shape=jax.ShapeDtypeStruct((EMBED_DIM, 1), jnp.float32),
        scratch_shapes=[
            pltpu.VMEM((CTX, EMBED_DIM, 128), jnp.float32),
            pltpu.SemaphoreType.DMA,
        ],
    )(idx, table_t)


def _logits_body(partials_ref, wt_ref, b_ref, out_ref):
    emb_col = partials_ref[...]
    prod = wt_ref[...] * emb_col
    out_ref[...] = jnp.sum(prod, axis=0) + b_ref[...]


def _logz_body(x_ref, logz_ref):
    x = x_ref[...]
    gidx = (lax.broadcasted_iota(jnp.int32, (ROWS2D, 128), 0) * 128
            + lax.broadcasted_iota(jnp.int32, (ROWS2D, 128), 1))
    x = jnp.where(gidx < VOCAB, x, NEG_BIG)
    m = jnp.max(x)
    s = jnp.sum(jnp.exp(x - m))
    logz_ref[0, 0] = m + jnp.log(s)


def _sub_body(logits_ref, logz_ref, out_ref):
    out_ref[...] = logits_ref[...] - logz_ref[0, 0]


def _tc_call(partials, wt, b):
    logits = pl.pallas_call(
        _logits_body,
        grid=(NBLK,),
        in_specs=[
            pl.BlockSpec((EMBED_DIM, 1), lambda i: (0, 0)),
            pl.BlockSpec((EMBED_DIM, VB), lambda i: (0, i)),
            pl.BlockSpec((VB,), lambda i: (i,)),
        ],
        out_specs=pl.BlockSpec((VB,), lambda i: (i,)),
        out_shape=jax.ShapeDtypeStruct((VPAD,), jnp.float32),
    )(partials, wt, b)
    logz = pl.pallas_call(
        _logz_body,
        in_specs=[pl.BlockSpec((ROWS2D, 128), lambda: (0, 0))],
        out_specs=pl.BlockSpec(memory_space=pltpu.SMEM),
        out_shape=jax.ShapeDtypeStruct((1, 1), jnp.float32),
    )(logits.reshape(ROWS2D, 128))
    return pl.pallas_call(
        _sub_body,
        grid=(NBLK3,),
        in_specs=[
            pl.BlockSpec((VB3,), lambda i: (i,)),
            pl.BlockSpec(memory_space=pltpu.SMEM),
        ],
        out_specs=pl.BlockSpec((VB3,), lambda i: (i,)),
        out_shape=jax.ShapeDtypeStruct((VOCAB,), jnp.float32),
    )(logits, logz)


def kernel(inputs, emb_table, W, b):
    partials = _tc_gather(inputs.astype(jnp.int32), emb_table.T)
    out = _tc_call(partials, W.T, b)
    return out.reshape(1, VOCAB)
